# baseline (speedup 1.0000x reference)
import dataclasses
import functools

import jax
import jax.numpy as jnp
from jax import lax
from jax.experimental import pallas as pl
from jax.experimental.pallas import tpu as pltpu
from jax.experimental.pallas import tpu_sc as plsc

_OUT_T, _OUT_H, _OUT_W = 2, 7, 7
_T_SCALE = 0.25
_S_SCALE = 0.25
_N, _C, _T, _H, _W = 4, 128, 8, 56, 56
_R = 128
_HW = _H * _W
_THW = _T * _HW
_NBIN = _OUT_T * _OUT_H * _OUT_W
_NSAMP = 4 * 14 * 14
_NCHUNK = _NSAMP // 16
_RPT = 4


def _sc_kernel_body(table, rois_hbm, out_hbm,
                    rois_v, idx_v, wts_v, bins_v, rows_v, acc_v,
                    sem, sem0, sem1):
    core = lax.axis_index("core")
    sub = lax.axis_index("subcore")
    wid = sub * 2 + core

    copy = pltpu.make_async_copy(rois_hbm, rois_v, sem)
    copy.start()
    copy.wait()

    lane = lax.iota(jnp.int32, 16)
    sems = (sem0, sem1)

    @pl.loop(0, _RPT)
    def _roi(rr):
        r = wid * _RPT + rr
        rv = rois_v[r]
        b = rv[0].astype(jnp.int32)
        base = b * _THW
        t1 = rv[1] * _T_SCALE
        y1 = rv[2] * _S_SCALE
        x1 = rv[3] * _S_SCALE
        t2 = rv[4] * _T_SCALE
        y2 = rv[5] * _S_SCALE
        x2 = rv[6] * _S_SCALE
        bt2 = jnp.maximum(t2 - t1, 1.0) * (0.5 / _OUT_T)
        bh2 = jnp.maximum(y2 - y1, 1.0) * (0.5 / _OUT_H)
        bw2 = jnp.maximum(x2 - x1, 1.0) * (0.5 / _OUT_W)

        @pl.loop(0, _NBIN)
        def _z(i):
            for c8 in range(8):
                acc_v[i, pl.ds(c8 * 16, 16)] = jnp.zeros((16,), jnp.float32)

        def stage(ch, buf):
            si = ch * 16 + lane
            ti = si // 196
            rem = si - ti * 196
            yi = rem // 14
            xi = rem - yi * 14

            tc = jnp.clip(t1 + (ti.astype(jnp.float32) + 0.5) * bt2,
                          0.0, float(_T - 1))
            yc = jnp.clip(y1 + (yi.astype(jnp.float32) + 0.5) * bh2,
                          0.0, float(_H - 1))
            xc = jnp.clip(x1 + (xi.astype(jnp.float32) + 0.5) * bw2,
                          0.0, float(_W - 1))
            t0 = tc.astype(jnp.int32)
            y0 = yc.astype(jnp.int32)
            x0 = xc.astype(jnp.int32)
            lt = tc - t0.astype(jnp.float32)
            ly = yc - y0.astype(jnp.float32)
            lx = xc - x0.astype(jnp.float32)
            ht = (1.0 - lt) * 0.125
            lt = lt * 0.125
            hy = 1.0 - ly
            hx = 1.0 - lx
            t1i = jnp.minimum(t0 + 1, _T - 1)
            y1i = jnp.minimum(y0 + 1, _H - 1)
            x1i = jnp.minimum(x0 + 1, _W - 1)

            r00 = base + t0 * _HW + y0 * _W + x0
            r01 = base + t0 * _HW + y1i * _W + x0
            r10 = base + t1i * _HW + y0 * _W + x0
            r11 = base + t1i * _HW + y1i * _W + x0
            corners = (
                (r00, ht * hy), (r01, ht * ly),
                (r10, lt * hy), (r11, lt * ly),
            )
            for k, (rid, wty) in enumerate(corners):
                idx_v[buf, pl.ds(k * 16, 16)] = rid
                wts_v[buf, pl.ds((2 * k) * 16, 16)] = wty * hx
                wts_v[buf, pl.ds((2 * k + 1) * 16, 16)] = wty * lx
            bins_v[buf, :] = (ti // 2) * (_OUT_H * _OUT_W) \
                + (yi // 2) * _OUT_W + (xi // 2)
            pltpu.make_async_copy(table.at[idx_v.at[buf]], rows_v.at[buf],
                                  sems[buf]).start()

        def combine(buf):
            pltpu.make_async_copy(table.at[idx_v.at[buf]], rows_v.at[buf],
                                  sems[buf]).wait()
            bvec = bins_v[buf, :]
            wvec = [wts_v[buf, pl.ds(k * 16, 16)] for k in range(8)]
            for p in range(8):
                bin_ = bvec[2 * p]
                vs = [acc_v[bin_, pl.ds(c8 * 16, 16)] for c8 in range(8)]
                for s in (2 * p, 2 * p + 1):
                    for k in range(4):
                        w0 = wvec[2 * k][s]
                        w1 = wvec[2 * k + 1][s]
                        for c32 in range(4):
                            u0 = rows_v[buf, k * 16 + s,
                                        pl.ds(c32 * 16, 16)]
                            u1 = rows_v[buf, k * 16 + s,
                                        pl.ds(64 + c32 * 16, 16)]
                            lo0 = plsc.bitcast(jnp.left_shift(u0, 16),
                                               jnp.float32)
                            hi0 = plsc.bitcast(
                                jnp.bitwise_and(u0, jnp.int32(-65536)),
                                jnp.float32)
                            lo1 = plsc.bitcast(jnp.left_shift(u1, 16),
                                               jnp.float32)
                            hi1 = plsc.bitcast(
                                jnp.bitwise_and(u1, jnp.int32(-65536)),
                                jnp.float32)
                            vs[c32 * 2] = vs[c32 * 2] + w0 * lo0 + w1 * lo1
                            vs[c32 * 2 + 1] = (vs[c32 * 2 + 1]
                                               + w0 * hi0 + w1 * hi1)
                for c8 in range(8):
                    acc_v[bin_, pl.ds(c8 * 16, 16)] = vs[c8]

        stage(0, 0)

        @pl.loop(0, _NCHUNK - 1, step=2)
        def _chunk(ch):
            stage(ch + 1, 1)
            combine(0)
            stage(ch + 2, 0)
            combine(1)

        combine(0)

        ocopy = pltpu.make_async_copy(acc_v, out_hbm.at[r], sem)
        ocopy.start()
        ocopy.wait()


@jax.jit
def kernel(features, rois):
    N, C, T, H, W = features.shape
    R = rois.shape[0]
    M = N * T * H * W
    tb = jnp.transpose(features, (0, 2, 3, 4, 1)).reshape(M, C).astype(
        jnp.bfloat16)
    tb2 = jnp.concatenate([tb, jnp.roll(tb, -1, axis=0)], axis=1)
    table = jax.lax.bitcast_convert_type(tb2.reshape(M, C, 2), jnp.int32)
    rois_p = jnp.pad(rois, ((0, 0), (0, 9)))

    mesh = plsc.VectorSubcoreMesh(core_axis_name="core",
                                  subcore_axis_name="subcore",
                                  num_cores=2, num_subcores=16)
    cp = pltpu.CompilerParams()
    if "needs_layout_passes" in pltpu.CompilerParams.__dataclass_fields__:
        cp = dataclasses.replace(cp, needs_layout_passes=False)
    sc = pl.kernel(
        _sc_kernel_body,
        out_type=jax.ShapeDtypeStruct((R, _NBIN, C), jnp.float32),
        mesh=mesh,
        scratch_types=[
            pltpu.VMEM((R, 16), jnp.float32),
            pltpu.VMEM((2, 64), jnp.int32),
            pltpu.VMEM((2, 128), jnp.float32),
            pltpu.VMEM((2, 16), jnp.int32),
            pltpu.VMEM((2, 64, C), jnp.int32),
            pltpu.VMEM((_NBIN, C), jnp.float32),
            pltpu.SemaphoreType.DMA,
            pltpu.SemaphoreType.DMA,
            pltpu.SemaphoreType.DMA,
        ],
        compiler_params=cp,
    )
    out = sc(table, rois_p)
    c = jnp.arange(C)
    cols = (c // 32) * 32 + (c % 2) * 16 + (c % 32) // 2
    out = out[:, :, cols]
    out = out.reshape(R, _OUT_T, _OUT_H, _OUT_W, C)
    return jnp.transpose(out, (0, 4, 1, 2, 3))


# speedup vs baseline: 2.1376x; 2.1376x over previous
import dataclasses
import functools

import jax
import jax.numpy as jnp
from jax import lax
from jax.experimental import pallas as pl
from jax.experimental.pallas import tpu as pltpu
from jax.experimental.pallas import tpu_sc as plsc

_OUT_T, _OUT_H, _OUT_W = 2, 7, 7
_T_SCALE = 0.25
_S_SCALE = 0.25
_N, _C, _T, _H, _W = 4, 128, 8, 56, 56
_R = 128
_HW = _H * _W
_THW = _T * _HW
_NBIN = _OUT_T * _OUT_H * _OUT_W
_NSAMP = 4 * 14 * 14
_NCHUNK = _NSAMP // 16
_RPT = 4


def _sc_kernel_body(table, rois_hbm, out_hbm,
                    rois_v, idx_v, wts_v, bins_v, rows_v, acc_v,
                    sem, sem0, sem1):
    core = lax.axis_index("core")
    sub = lax.axis_index("subcore")
    wid = sub * 2 + core

    copy = pltpu.make_async_copy(rois_hbm, rois_v, sem)
    copy.start()
    copy.wait()

    lane = lax.iota(jnp.int32, 16)
    sems = (sem0, sem1)

    @pl.loop(0, _RPT)
    def _roi(rr):
        r = wid * _RPT + rr
        rv = rois_v[r]
        b = rv[0].astype(jnp.int32)
        base = b * _THW
        t1 = rv[1] * _T_SCALE
        y1 = rv[2] * _S_SCALE
        x1 = rv[3] * _S_SCALE
        t2 = rv[4] * _T_SCALE
        y2 = rv[5] * _S_SCALE
        x2 = rv[6] * _S_SCALE
        bt2 = jnp.maximum(t2 - t1, 1.0) * (0.5 / _OUT_T)
        bh2 = jnp.maximum(y2 - y1, 1.0) * (0.5 / _OUT_H)
        bw2 = jnp.maximum(x2 - x1, 1.0) * (0.5 / _OUT_W)

        @pl.loop(0, _NBIN)
        def _z(i):
            for c8 in range(8):
                acc_v[i, pl.ds(c8 * 16, 16)] = jnp.zeros((16,), jnp.float32)

        def stage(ch, buf):
            si = ch * 16 + lane
            ti = si // 196
            rem = si - ti * 196
            yi = rem // 14
            xi = rem - yi * 14

            tc = jnp.clip(t1 + (ti.astype(jnp.float32) + 0.5) * bt2,
                          0.0, float(_T - 1))
            yc = jnp.clip(y1 + (yi.astype(jnp.float32) + 0.5) * bh2,
                          0.0, float(_H - 1))
            xc = jnp.clip(x1 + (xi.astype(jnp.float32) + 0.5) * bw2,
                          0.0, float(_W - 1))
            t0 = tc.astype(jnp.int32)
            y0 = yc.astype(jnp.int32)
            x0 = xc.astype(jnp.int32)
            lt = tc - t0.astype(jnp.float32)
            ly = yc - y0.astype(jnp.float32)
            lx = xc - x0.astype(jnp.float32)
            ht = (1.0 - lt) * 0.125
            lt = lt * 0.125
            hy = 1.0 - ly
            hx = 1.0 - lx
            t1i = jnp.minimum(t0 + 1, _T - 1)
            y1i = jnp.minimum(y0 + 1, _H - 1)
            x1i = jnp.minimum(x0 + 1, _W - 1)

            r00 = base + t0 * _HW + y0 * _W
            r01 = base + t0 * _HW + y1i * _W
            r10 = base + t1i * _HW + y0 * _W
            r11 = base + t1i * _HW + y1i * _W
            corners = (
                (r00 + x0, ht * hy * hx), (r00 + x1i, ht * hy * lx),
                (r01 + x0, ht * ly * hx), (r01 + x1i, ht * ly * lx),
                (r10 + x0, lt * hy * hx), (r10 + x1i, lt * hy * lx),
                (r11 + x0, lt * ly * hx), (r11 + x1i, lt * ly * lx),
            )
            for k, (rid, wv) in enumerate(corners):
                idx_v[buf, pl.ds(k * 16, 16)] = rid
                wts_v[buf, pl.ds(k * 16, 16)] = wv
            bins_v[buf, :] = (ti // 2) * (_OUT_H * _OUT_W) \
                + (yi // 2) * _OUT_W + (xi // 2)
            pltpu.make_async_copy(table.at[idx_v.at[buf]], rows_v.at[buf],
                                  sems[buf]).start()

        def combine(buf):
            pltpu.make_async_copy(table.at[idx_v.at[buf]], rows_v.at[buf],
                                  sems[buf]).wait()
            bvec = bins_v[buf, :]
            wvec = [wts_v[buf, pl.ds(k * 16, 16)] for k in range(8)]
            for p in range(8):
                bin_ = bvec[2 * p]
                vs = [acc_v[bin_, pl.ds(c8 * 16, 16)] for c8 in range(8)]
                for s in (2 * p, 2 * p + 1):
                    for k in range(8):
                        w = wvec[k][s]
                        for c8 in range(8):
                            vs[c8] = vs[c8] + w * rows_v[buf, k * 16 + s,
                                                         pl.ds(c8 * 16, 16)]
                for c8 in range(8):
                    acc_v[bin_, pl.ds(c8 * 16, 16)] = vs[c8]

        stage(0, 0)

        @pl.loop(0, _NCHUNK - 1, step=2)
        def _chunk(ch):
            stage(ch + 1, 1)
            combine(0)
            stage(ch + 2, 0)
            combine(1)

        combine(0)

        ocopy = pltpu.make_async_copy(acc_v, out_hbm.at[r], sem)
        ocopy.start()
        ocopy.wait()


@jax.jit
def kernel(features, rois):
    N, C, T, H, W = features.shape
    R = rois.shape[0]
    table = jnp.transpose(features, (0, 2, 3, 4, 1)).reshape(N * T * H * W, C)
    rois_p = jnp.pad(rois, ((0, 0), (0, 9)))

    mesh = plsc.VectorSubcoreMesh(core_axis_name="core",
                                  subcore_axis_name="subcore",
                                  num_cores=2, num_subcores=16)
    cp = pltpu.CompilerParams()
    if "needs_layout_passes" in pltpu.CompilerParams.__dataclass_fields__:
        cp = dataclasses.replace(cp, needs_layout_passes=False)
    sc = pl.kernel(
        _sc_kernel_body,
        out_type=jax.ShapeDtypeStruct((R, _NBIN, C), jnp.float32),
        mesh=mesh,
        scratch_types=[
            pltpu.VMEM((R, 16), jnp.float32),
            pltpu.VMEM((2, 128), jnp.int32),
            pltpu.VMEM((2, 128), jnp.float32),
            pltpu.VMEM((2, 16), jnp.int32),
            pltpu.VMEM((2, 128, C), jnp.float32),
            pltpu.VMEM((_NBIN, C), jnp.float32),
            pltpu.SemaphoreType.DMA,
            pltpu.SemaphoreType.DMA,
            pltpu.SemaphoreType.DMA,
        ],
        compiler_params=cp,
    )
    out = sc(table, rois_p)
    out = out.reshape(R, _OUT_T, _OUT_H, _OUT_W, C)
    return jnp.transpose(out, (0, 4, 1, 2, 3))
